# Initial kernel scaffold; baseline (speedup 1.0000x reference)
import jax
import jax.numpy as jnp
from jax.experimental import pallas as pl


def kernel(hidden_states, Wr, br, W1, b1, W2, b2, Wq, bq, Wk, bk, Wv, bv, Wo, bo):
    raise NotImplementedError("write your pallas kernel here")


# speedup vs baseline: 1.6027x; 1.6027x over previous
import functools

import jax
import jax.numpy as jnp
from jax.experimental import pallas as pl
from jax.experimental.pallas import tpu as pltpu

_INTERPRET = False


def _router_body(x_ref, wr_ref, br_ref, idx_ref, w_ref):
    s = x_ref.shape[0]
    pooled = jnp.sum(x_ref[...], axis=0, keepdims=True) * (1.0 / s)
    logits = jnp.dot(pooled, wr_ref[...], preferred_element_type=jnp.float32)
    logits = logits + br_ref[...]
    m = jnp.max(logits, axis=-1, keepdims=True)
    e = jnp.exp(logits - m)
    probs = e / jnp.sum(e, axis=-1, keepdims=True)
    t = probs.shape[-1]
    iota = jax.lax.broadcasted_iota(jnp.int32, (1, t), 1)
    big = jnp.int32(2**30)
    v0 = jnp.max(probs)
    i0 = jnp.min(jnp.where(probs == v0, iota, big))
    masked = jnp.where(iota == i0, -jnp.inf, probs)
    v1 = jnp.max(masked)
    i1 = jnp.min(jnp.where(masked == v1, iota, big))
    e1 = jnp.exp(v1 - v0)
    w0 = 1.0 / (1.0 + e1)
    idx_ref[0] = i0
    idx_ref[1] = i1
    w_ref[0] = w0
    w_ref[1] = 1.0 - w0


def _router(x, Wr, br):
    t = Wr.shape[1]
    return pl.pallas_call(
        _router_body,
        out_shape=(
            jax.ShapeDtypeStruct((2,), jnp.int32),
            jax.ShapeDtypeStruct((2,), jnp.float32),
        ),
        in_specs=[
            pl.BlockSpec(memory_space=pltpu.VMEM),
            pl.BlockSpec(memory_space=pltpu.VMEM),
            pl.BlockSpec(memory_space=pltpu.VMEM),
        ],
        out_specs=(
            pl.BlockSpec(memory_space=pltpu.SMEM),
            pl.BlockSpec(memory_space=pltpu.SMEM),
        ),
        interpret=_INTERPRET,
    )(x, Wr, br.reshape(1, t))


def _ffn_body(idx_ref, x_ref, w1_ref, b1_ref, w2_ref, b2_ref, wqkv_ref,
              bqkv_ref, out_ref, acc_ref):
    f = pl.program_id(2)
    nf = pl.num_programs(2)
    h1 = jnp.dot(x_ref[...], w1_ref[0], preferred_element_type=jnp.float32)
    h1 = jax.nn.gelu(h1 + b1_ref[0])
    part = jnp.dot(h1, w2_ref[0], preferred_element_type=jnp.float32)

    @pl.when(f == 0)
    def _():
        acc_ref[...] = part + b2_ref[0]

    @pl.when(f > 0)
    def _():
        acc_ref[...] += part

    @pl.when(f == nf - 1)
    def _():
        out_ref[0] = (
            jnp.dot(acc_ref[...], wqkv_ref[...],
                    preferred_element_type=jnp.float32)
            + bqkv_ref[...]
        )


def _ffn_qkv(x, top_idx, W1, b1, W2, b2, Wqkv, bqkv, *, bs, fblk):
    s, h = x.shape
    t, _, ff = W1.shape
    k = top_idx.shape[0]
    grid = (k, s // bs, ff // fblk)
    h3 = Wqkv.shape[1]
    gs = pltpu.PrefetchScalarGridSpec(
        num_scalar_prefetch=1,
        grid=grid,
        in_specs=[
            pl.BlockSpec((bs, h), lambda kk, ss, ffi, idx: (ss, 0)),
            pl.BlockSpec((1, h, fblk), lambda kk, ss, ffi, idx: (idx[kk], 0, ffi)),
            pl.BlockSpec((1, 1, fblk), lambda kk, ss, ffi, idx: (idx[kk], 0, ffi)),
            pl.BlockSpec((1, fblk, h), lambda kk, ss, ffi, idx: (idx[kk], ffi, 0)),
            pl.BlockSpec((1, 1, h), lambda kk, ss, ffi, idx: (idx[kk], 0, 0)),
            pl.BlockSpec((h, h3), lambda kk, ss, ffi, idx: (0, 0)),
            pl.BlockSpec((1, h3), lambda kk, ss, ffi, idx: (0, 0)),
        ],
        out_specs=pl.BlockSpec((1, bs, h3), lambda kk, ss, ffi, idx: (kk, ss, 0)),
        scratch_shapes=[pltpu.VMEM((bs, h), jnp.float32)],
    )
    return pl.pallas_call(
        _ffn_body,
        grid_spec=gs,
        out_shape=jax.ShapeDtypeStruct((k, s, h3), jnp.float32),
        interpret=_INTERPRET,
    )(top_idx, x, W1, b1.reshape(t, 1, ff), W2, b2.reshape(t, 1, h),
      Wqkv, bqkv.reshape(1, h3))


def _attn_body(q_ref, k_ref, v_ref, o_ref, *, scale):
    s = jax.lax.dot_general(
        q_ref[...], k_ref[...], (((1,), (1,)), ((), ())),
        preferred_element_type=jnp.float32,
    ) * scale
    m = jnp.max(s, axis=-1, keepdims=True)
    p = jnp.exp(s - m)
    l = jnp.sum(p, axis=-1, keepdims=True)
    ctx = jnp.dot(p, v_ref[...], preferred_element_type=jnp.float32)
    o_ref[...] = ctx / l


def _attention(qkv, *, nh, dh, qbs):
    sk, h3 = qkv.shape
    h = h3 // 3
    grid = (nh, sk // qbs)
    body = functools.partial(_attn_body, scale=1.0 / (dh ** 0.5))
    return pl.pallas_call(
        body,
        grid=grid,
        in_specs=[
            pl.BlockSpec((qbs, dh), lambda hh, qb: (qb, hh)),
            pl.BlockSpec((sk, dh), lambda hh, qb: (0, nh + hh)),
            pl.BlockSpec((sk, dh), lambda hh, qb: (0, 2 * nh + hh)),
        ],
        out_specs=pl.BlockSpec((qbs, dh), lambda hh, qb: (qb, hh)),
        out_shape=jax.ShapeDtypeStruct((sk, h), jnp.float32),
        interpret=_INTERPRET,
    )(qkv, qkv, qkv)


def _combine_body(w_ref, c0_ref, c1_ref, wo_ref, bo_ref, o_ref):
    mixed = w_ref[0] * c0_ref[0] + w_ref[1] * c1_ref[0]
    o_ref[...] = (
        jnp.dot(mixed, wo_ref[...], preferred_element_type=jnp.float32)
        + bo_ref[...]
    )


def _combine(ctx, w, Wo, bo, *, bs):
    k, s, h = ctx.shape
    grid = (s // bs,)
    return pl.pallas_call(
        _combine_body,
        grid=grid,
        in_specs=[
            pl.BlockSpec(memory_space=pltpu.SMEM),
            pl.BlockSpec((1, bs, h), lambda ss: (0, ss, 0)),
            pl.BlockSpec((1, bs, h), lambda ss: (1, ss, 0)),
            pl.BlockSpec((h, h), lambda ss: (0, 0)),
            pl.BlockSpec((1, h), lambda ss: (0, 0)),
        ],
        out_specs=pl.BlockSpec((bs, h), lambda ss: (ss, 0)),
        out_shape=jax.ShapeDtypeStruct((s, h), jnp.float32),
        interpret=_INTERPRET,
    )(w, ctx, ctx, Wo, bo.reshape(1, h))


def kernel(hidden_states, Wr, br, W1, b1, W2, b2, Wq, bq, Wk, bk, Wv, bv,
           Wo, bo):
    b_, s_, h_ = hidden_states.shape
    t, _, ff = W1.shape
    nh = 8
    dh = h_ // nh
    x = hidden_states.reshape(s_, h_)

    top_idx, w = _router(x, Wr, br)

    Wqkv = jnp.concatenate([Wq, Wk, Wv], axis=1)
    bqkv = jnp.concatenate([bq, bk, bv], axis=0)
    qkv = _ffn_qkv(x, top_idx, W1, b1, W2, b2, Wqkv, bqkv, bs=512, fblk=1024)

    k_sel = top_idx.shape[0]
    ctx = _attention(qkv.reshape(k_sel * s_, 3 * h_), nh=nh, dh=dh, qbs=512)

    out = _combine(ctx.reshape(k_sel, s_, h_), w, Wo, bo, bs=512)
    return out.reshape(b_, s_, h_)


# speedup vs baseline: 1.6219x; 1.0120x over previous
import functools

import jax
import jax.numpy as jnp
from jax.experimental import pallas as pl
from jax.experimental.pallas import tpu as pltpu

_INTERPRET = False


def _router_body(x_ref, wr_ref, br_ref, idx_ref, w_ref):
    s = x_ref.shape[0]
    pooled = jnp.sum(x_ref[...], axis=0, keepdims=True) * (1.0 / s)
    logits = jnp.dot(pooled, wr_ref[...], preferred_element_type=jnp.float32)
    logits = logits + br_ref[...]
    m = jnp.max(logits, axis=-1, keepdims=True)
    e = jnp.exp(logits - m)
    probs = e / jnp.sum(e, axis=-1, keepdims=True)
    t = probs.shape[-1]
    iota = jax.lax.broadcasted_iota(jnp.int32, (1, t), 1)
    big = jnp.int32(2**30)
    v0 = jnp.max(probs)
    i0 = jnp.min(jnp.where(probs == v0, iota, big))
    masked = jnp.where(iota == i0, -jnp.inf, probs)
    v1 = jnp.max(masked)
    i1 = jnp.min(jnp.where(masked == v1, iota, big))
    e1 = jnp.exp(v1 - v0)
    w0 = 1.0 / (1.0 + e1)
    idx_ref[0] = i0
    idx_ref[1] = i1
    w_ref[0] = w0
    w_ref[1] = 1.0 - w0


def _router(x, Wr, br):
    t = Wr.shape[1]
    return pl.pallas_call(
        _router_body,
        out_shape=(
            jax.ShapeDtypeStruct((2,), jnp.int32),
            jax.ShapeDtypeStruct((2,), jnp.float32),
        ),
        in_specs=[
            pl.BlockSpec(memory_space=pltpu.VMEM),
            pl.BlockSpec(memory_space=pltpu.VMEM),
            pl.BlockSpec(memory_space=pltpu.VMEM),
        ],
        out_specs=(
            pl.BlockSpec(memory_space=pltpu.SMEM),
            pl.BlockSpec(memory_space=pltpu.SMEM),
        ),
        interpret=_INTERPRET,
    )(x, Wr, br.reshape(1, t))


def _ffn_body(idx_ref, x_ref, w1_ref, b1_ref, w2_ref, b2_ref, wqkv_ref,
              bqkv_ref, out_ref, acc_ref):
    f = pl.program_id(2)
    nf = pl.num_programs(2)
    h1 = jnp.dot(x_ref[...], w1_ref[0], preferred_element_type=jnp.float32)
    h1 = jax.nn.gelu(h1 + b1_ref[0])
    part = jnp.dot(h1.astype(jnp.bfloat16), w2_ref[0],
                   preferred_element_type=jnp.float32)

    @pl.when(f == 0)
    def _():
        acc_ref[...] = part + b2_ref[0]

    @pl.when(f > 0)
    def _():
        acc_ref[...] += part

    @pl.when(f == nf - 1)
    def _():
        out_ref[0] = (
            jnp.dot(acc_ref[...], wqkv_ref[...],
                    preferred_element_type=jnp.float32)
            + bqkv_ref[...]
        ).astype(jnp.bfloat16)


def _ffn_qkv(x, top_idx, W1, b1, W2, b2, Wqkv, bqkv, *, bs, fblk):
    s, h = x.shape
    t, _, ff = W1.shape
    k = top_idx.shape[0]
    grid = (k, s // bs, ff // fblk)
    h3 = Wqkv.shape[1]
    gs = pltpu.PrefetchScalarGridSpec(
        num_scalar_prefetch=1,
        grid=grid,
        in_specs=[
            pl.BlockSpec((bs, h), lambda kk, ss, ffi, idx: (ss, 0)),
            pl.BlockSpec((1, h, fblk), lambda kk, ss, ffi, idx: (idx[kk], 0, ffi)),
            pl.BlockSpec((1, 1, fblk), lambda kk, ss, ffi, idx: (idx[kk], 0, ffi)),
            pl.BlockSpec((1, fblk, h), lambda kk, ss, ffi, idx: (idx[kk], ffi, 0)),
            pl.BlockSpec((1, 1, h), lambda kk, ss, ffi, idx: (idx[kk], 0, 0)),
            pl.BlockSpec((h, h3), lambda kk, ss, ffi, idx: (0, 0)),
            pl.BlockSpec((1, h3), lambda kk, ss, ffi, idx: (0, 0)),
        ],
        out_specs=pl.BlockSpec((1, bs, h3), lambda kk, ss, ffi, idx: (kk, ss, 0)),
        scratch_shapes=[pltpu.VMEM((bs, h), jnp.float32)],
    )
    return pl.pallas_call(
        _ffn_body,
        grid_spec=gs,
        out_shape=jax.ShapeDtypeStruct((k, s, h3), jnp.bfloat16),
        interpret=_INTERPRET,
    )(top_idx, x, W1, b1.reshape(t, 1, ff), W2, b2.reshape(t, 1, h),
      Wqkv, bqkv.reshape(1, h3))


def _attn_body(q_ref, k_ref, v_ref, o_ref):
    s = jax.lax.dot_general(
        q_ref[...], k_ref[...], (((1,), (1,)), ((), ())),
        preferred_element_type=jnp.float32,
    )
    m = jnp.max(s, axis=-1, keepdims=True)
    p = jnp.exp(s - m).astype(jnp.bfloat16)
    l = jnp.sum(p.astype(jnp.float32), axis=-1, keepdims=True)
    ctx = jnp.dot(p, v_ref[...], preferred_element_type=jnp.float32)
    o_ref[...] = (ctx / l).astype(jnp.bfloat16)


def _attention(qkv, *, nh, dh, qbs):
    sk, h3 = qkv.shape
    h = h3 // 3
    grid = (nh, sk // qbs)
    return pl.pallas_call(
        _attn_body,
        grid=grid,
        in_specs=[
            pl.BlockSpec((qbs, dh), lambda hh, qb: (qb, hh)),
            pl.BlockSpec((sk, dh), lambda hh, qb: (0, nh + hh)),
            pl.BlockSpec((sk, dh), lambda hh, qb: (0, 2 * nh + hh)),
        ],
        out_specs=pl.BlockSpec((qbs, dh), lambda hh, qb: (qb, hh)),
        out_shape=jax.ShapeDtypeStruct((sk, h), jnp.bfloat16),
        interpret=_INTERPRET,
    )(qkv, qkv, qkv)


def _combine_body(w_ref, c0_ref, c1_ref, wo_ref, bo_ref, o_ref):
    mixed = w_ref[0] * c0_ref[0] + w_ref[1] * c1_ref[0]
    o_ref[...] = (
        jnp.dot(mixed, wo_ref[...], preferred_element_type=jnp.float32)
        + bo_ref[...]
    )


def _combine(ctx, w, Wo, bo, *, bs):
    k, s, h = ctx.shape
    grid = (s // bs,)
    return pl.pallas_call(
        _combine_body,
        grid=grid,
        in_specs=[
            pl.BlockSpec(memory_space=pltpu.SMEM),
            pl.BlockSpec((1, bs, h), lambda ss: (0, ss, 0)),
            pl.BlockSpec((1, bs, h), lambda ss: (1, ss, 0)),
            pl.BlockSpec((h, h), lambda ss: (0, 0)),
            pl.BlockSpec((1, h), lambda ss: (0, 0)),
        ],
        out_specs=pl.BlockSpec((bs, h), lambda ss: (ss, 0)),
        out_shape=jax.ShapeDtypeStruct((s, h), jnp.float32),
        interpret=_INTERPRET,
    )(w, ctx, ctx, Wo, bo.reshape(1, h))


def kernel(hidden_states, Wr, br, W1, b1, W2, b2, Wq, bq, Wk, bk, Wv, bv,
           Wo, bo):
    b_, s_, h_ = hidden_states.shape
    t, _, ff = W1.shape
    nh = 8
    dh = h_ // nh
    x = hidden_states.reshape(s_, h_)

    top_idx, w = _router(x, Wr, br)

    scale = 1.0 / (dh ** 0.5)
    Wqkv = jnp.concatenate([Wq * scale, Wk, Wv], axis=1)
    bqkv = jnp.concatenate([bq * scale, bk, bv], axis=0)
    qkv = _ffn_qkv(x, top_idx, W1, b1, W2, b2, Wqkv, bqkv, bs=512, fblk=1024)

    k_sel = top_idx.shape[0]
    ctx = _attention(qkv.reshape(k_sel * s_, 3 * h_), nh=nh, dh=dh, qbs=512)

    out = _combine(ctx.reshape(k_sel, s_, h_), w, Wo, bo, bs=512)
    return out.reshape(b_, s_, h_)


# speedup vs baseline: 1.6609x; 1.0240x over previous
import functools

import jax
import jax.numpy as jnp
from jax.experimental import pallas as pl
from jax.experimental.pallas import tpu as pltpu

_INTERPRET = False


def _router_body(x_ref, wr_ref, br_ref, idx_ref, w_ref):
    s = x_ref.shape[0]
    pooled = jnp.sum(x_ref[...], axis=0, keepdims=True) * (1.0 / s)
    logits = jnp.dot(pooled, wr_ref[...], preferred_element_type=jnp.float32)
    logits = logits + br_ref[...]
    m = jnp.max(logits, axis=-1, keepdims=True)
    e = jnp.exp(logits - m)
    probs = e / jnp.sum(e, axis=-1, keepdims=True)
    t = probs.shape[-1]
    iota = jax.lax.broadcasted_iota(jnp.int32, (1, t), 1)
    big = jnp.int32(2**30)
    v0 = jnp.max(probs)
    i0 = jnp.min(jnp.where(probs == v0, iota, big))
    masked = jnp.where(iota == i0, -jnp.inf, probs)
    v1 = jnp.max(masked)
    i1 = jnp.min(jnp.where(masked == v1, iota, big))
    e1 = jnp.exp(v1 - v0)
    w0 = 1.0 / (1.0 + e1)
    idx_ref[0] = i0
    idx_ref[1] = i1
    w_ref[0] = w0
    w_ref[1] = 1.0 - w0


def _router(x, Wr, br):
    t = Wr.shape[1]
    return pl.pallas_call(
        _router_body,
        out_shape=(
            jax.ShapeDtypeStruct((2,), jnp.int32),
            jax.ShapeDtypeStruct((2,), jnp.float32),
        ),
        in_specs=[
            pl.BlockSpec(memory_space=pltpu.VMEM),
            pl.BlockSpec(memory_space=pltpu.VMEM),
            pl.BlockSpec(memory_space=pltpu.VMEM),
        ],
        out_specs=(
            pl.BlockSpec(memory_space=pltpu.SMEM),
            pl.BlockSpec(memory_space=pltpu.SMEM),
        ),
        interpret=_INTERPRET,
    )(x, Wr, br.reshape(1, t))


def _ffn_body(idx_ref, x_ref, w1_ref, b1_ref, w2_ref, b2_ref, wqkv_ref,
              bqkv_ref, out_ref, acc_ref):
    f = pl.program_id(2)
    nf = pl.num_programs(2)
    h1 = jnp.dot(x_ref[...], w1_ref[0], preferred_element_type=jnp.float32)
    h1 = jax.nn.gelu(h1 + b1_ref[0])
    part = jnp.dot(h1.astype(jnp.bfloat16), w2_ref[0],
                   preferred_element_type=jnp.float32)

    @pl.when(f == 0)
    def _():
        acc_ref[...] = part + b2_ref[0]

    @pl.when(f > 0)
    def _():
        acc_ref[...] += part

    @pl.when(f == nf - 1)
    def _():
        out_ref[0] = (
            jnp.dot(acc_ref[...], wqkv_ref[...],
                    preferred_element_type=jnp.float32)
            + bqkv_ref[...]
        ).astype(jnp.bfloat16)


def _ffn_qkv(x, top_idx, W1, b1, W2, b2, Wqkv, bqkv, *, bs, fblk):
    s, h = x.shape
    t, _, ff = W1.shape
    k = top_idx.shape[0]
    grid = (k, s // bs, ff // fblk)
    h3 = Wqkv.shape[1]
    gs = pltpu.PrefetchScalarGridSpec(
        num_scalar_prefetch=1,
        grid=grid,
        in_specs=[
            pl.BlockSpec((bs, h), lambda kk, ss, ffi, idx: (ss, 0)),
            pl.BlockSpec((1, h, fblk), lambda kk, ss, ffi, idx: (idx[kk], 0, ffi)),
            pl.BlockSpec((1, 1, fblk), lambda kk, ss, ffi, idx: (idx[kk], 0, ffi)),
            pl.BlockSpec((1, fblk, h), lambda kk, ss, ffi, idx: (idx[kk], ffi, 0)),
            pl.BlockSpec((1, 1, h), lambda kk, ss, ffi, idx: (idx[kk], 0, 0)),
            pl.BlockSpec((h, h3), lambda kk, ss, ffi, idx: (0, 0)),
            pl.BlockSpec((1, h3), lambda kk, ss, ffi, idx: (0, 0)),
        ],
        out_specs=pl.BlockSpec((1, bs, h3), lambda kk, ss, ffi, idx: (kk, ss, 0)),
        scratch_shapes=[pltpu.VMEM((bs, h), jnp.float32)],
    )
    return pl.pallas_call(
        _ffn_body,
        grid_spec=gs,
        out_shape=jax.ShapeDtypeStruct((k, s, h3), jnp.bfloat16),
        interpret=_INTERPRET,
    )(top_idx, x, W1, b1.reshape(t, 1, ff), W2, b2.reshape(t, 1, h),
      Wqkv, bqkv.reshape(1, h3))


def _attn_body(q_ref, k_ref, v_ref, o_ref):
    s = jax.lax.dot_general(
        q_ref[...], k_ref[...], (((1,), (1,)), ((), ())),
        preferred_element_type=jnp.float32,
    )
    m = jnp.max(s, axis=-1, keepdims=True)
    p = jnp.exp(s - m).astype(jnp.bfloat16)
    l = jnp.sum(p.astype(jnp.float32), axis=-1, keepdims=True)
    ctx = jnp.dot(p, v_ref[...], preferred_element_type=jnp.float32)
    o_ref[...] = (ctx / l).astype(jnp.bfloat16)


def _attention(qkv, *, nh, dh, qbs):
    sk, h3 = qkv.shape
    h = h3 // 3
    grid = (nh, sk // qbs)
    return pl.pallas_call(
        _attn_body,
        grid=grid,
        in_specs=[
            pl.BlockSpec((qbs, dh), lambda hh, qb: (qb, hh)),
            pl.BlockSpec((sk, dh), lambda hh, qb: (0, nh + hh)),
            pl.BlockSpec((sk, dh), lambda hh, qb: (0, 2 * nh + hh)),
        ],
        out_specs=pl.BlockSpec((qbs, dh), lambda hh, qb: (qb, hh)),
        out_shape=jax.ShapeDtypeStruct((sk, h), jnp.bfloat16),
        interpret=_INTERPRET,
    )(qkv, qkv, qkv)


def _combine_body(w_ref, c0_ref, c1_ref, wo_ref, bo_ref, o_ref):
    mixed = w_ref[0] * c0_ref[0] + w_ref[1] * c1_ref[0]
    o_ref[...] = (
        jnp.dot(mixed, wo_ref[...], preferred_element_type=jnp.float32)
        + bo_ref[...]
    )


def _combine(ctx, w, Wo, bo, *, bs):
    k, s, h = ctx.shape
    grid = (s // bs,)
    return pl.pallas_call(
        _combine_body,
        grid=grid,
        in_specs=[
            pl.BlockSpec(memory_space=pltpu.SMEM),
            pl.BlockSpec((1, bs, h), lambda ss: (0, ss, 0)),
            pl.BlockSpec((1, bs, h), lambda ss: (1, ss, 0)),
            pl.BlockSpec((h, h), lambda ss: (0, 0)),
            pl.BlockSpec((1, h), lambda ss: (0, 0)),
        ],
        out_specs=pl.BlockSpec((bs, h), lambda ss: (ss, 0)),
        out_shape=jax.ShapeDtypeStruct((s, h), jnp.float32),
        interpret=_INTERPRET,
    )(w, ctx, ctx, Wo, bo.reshape(1, h))


def kernel(hidden_states, Wr, br, W1, b1, W2, b2, Wq, bq, Wk, bk, Wv, bv,
           Wo, bo):
    b_, s_, h_ = hidden_states.shape
    t, _, ff = W1.shape
    nh = 8
    dh = h_ // nh
    x = hidden_states.reshape(s_, h_)

    top_idx, w = _router(x, Wr, br)

    scale = 1.0 / (dh ** 0.5)
    Wqkv = jnp.concatenate([Wq * scale, Wk, Wv], axis=1)
    bqkv = jnp.concatenate([bq * scale, bk, bv], axis=0)
    qkv = _ffn_qkv(x, top_idx, W1, b1, W2, b2, Wqkv, bqkv, bs=512, fblk=1024)

    k_sel = top_idx.shape[0]
    ctx = _attention(qkv.reshape(k_sel * s_, 3 * h_), nh=nh, dh=dh, qbs=1024)

    out = _combine(ctx.reshape(k_sel, s_, h_), w, Wo, bo, bs=512)
    return out.reshape(b_, s_, h_)


# speedup vs baseline: 1.6767x; 1.0096x over previous
import functools

import jax
import jax.numpy as jnp
import numpy as np
from jax.experimental import pallas as pl
from jax.experimental.pallas import tpu as pltpu

_INTERPRET = False


def _router_body(x_ref, wr_ref, br_ref, idx_ref, w_ref):
    s = x_ref.shape[0]
    pooled = jnp.sum(x_ref[...], axis=0, keepdims=True) * (1.0 / s)
    logits = jnp.dot(pooled, wr_ref[...], preferred_element_type=jnp.float32)
    logits = logits + br_ref[...]
    m = jnp.max(logits, axis=-1, keepdims=True)
    e = jnp.exp(logits - m)
    probs = e / jnp.sum(e, axis=-1, keepdims=True)
    t = probs.shape[-1]
    iota = jax.lax.broadcasted_iota(jnp.int32, (1, t), 1)
    big = jnp.int32(2**30)
    v0 = jnp.max(probs)
    i0 = jnp.min(jnp.where(probs == v0, iota, big))
    masked = jnp.where(iota == i0, -jnp.inf, probs)
    v1 = jnp.max(masked)
    i1 = jnp.min(jnp.where(masked == v1, iota, big))
    e1 = jnp.exp(v1 - v0)
    w0 = 1.0 / (1.0 + e1)
    idx_ref[0] = i0
    idx_ref[1] = i1
    w_ref[0] = w0
    w_ref[1] = 1.0 - w0


def _router(x, Wr, br):
    t = Wr.shape[1]
    return pl.pallas_call(
        _router_body,
        out_shape=(
            jax.ShapeDtypeStruct((2,), jnp.int32),
            jax.ShapeDtypeStruct((2,), jnp.float32),
        ),
        in_specs=[
            pl.BlockSpec(memory_space=pltpu.VMEM),
            pl.BlockSpec(memory_space=pltpu.VMEM),
            pl.BlockSpec(memory_space=pltpu.VMEM),
        ],
        out_specs=(
            pl.BlockSpec(memory_space=pltpu.SMEM),
            pl.BlockSpec(memory_space=pltpu.SMEM),
        ),
        interpret=_INTERPRET,
    )(x, Wr, br.reshape(1, t))


def _ffn_body(idx_ref, x_ref, w1_ref, b1_ref, w2_ref, b2_ref, wqkv_ref,
              bqkv_ref, out_ref, acc_ref):
    f = pl.program_id(2)
    nf = pl.num_programs(2)
    h1 = jnp.dot(x_ref[...], w1_ref[0], preferred_element_type=jnp.float32)
    h1 = jax.nn.gelu(h1 + b1_ref[0])
    part = jnp.dot(h1.astype(jnp.bfloat16), w2_ref[0],
                   preferred_element_type=jnp.float32)

    @pl.when(f == 0)
    def _():
        acc_ref[...] = part + b2_ref[0]

    @pl.when(f > 0)
    def _():
        acc_ref[...] += part

    @pl.when(f == nf - 1)
    def _():
        out_ref[0] = (
            jnp.dot(acc_ref[...], wqkv_ref[...],
                    preferred_element_type=jnp.float32)
            + bqkv_ref[...]
        ).astype(jnp.bfloat16)


def _ffn_qkv(x, top_idx, W1, b1, W2, b2, Wqkv, bqkv, *, bs, fblk):
    s, h = x.shape
    t, _, ff = W1.shape
    k = top_idx.shape[0]
    grid = (k, s // bs, ff // fblk)
    h3 = Wqkv.shape[1]
    gs = pltpu.PrefetchScalarGridSpec(
        num_scalar_prefetch=1,
        grid=grid,
        in_specs=[
            pl.BlockSpec((bs, h), lambda kk, ss, ffi, idx: (ss, 0)),
            pl.BlockSpec((1, h, fblk), lambda kk, ss, ffi, idx: (idx[kk], 0, ffi)),
            pl.BlockSpec((1, 1, fblk), lambda kk, ss, ffi, idx: (idx[kk], 0, ffi)),
            pl.BlockSpec((1, fblk, h), lambda kk, ss, ffi, idx: (idx[kk], ffi, 0)),
            pl.BlockSpec((1, 1, h), lambda kk, ss, ffi, idx: (idx[kk], 0, 0)),
            pl.BlockSpec((h, h3), lambda kk, ss, ffi, idx: (0, 0)),
            pl.BlockSpec((1, h3), lambda kk, ss, ffi, idx: (0, 0)),
        ],
        out_specs=pl.BlockSpec((1, bs, h3), lambda kk, ss, ffi, idx: (kk, ss, 0)),
        scratch_shapes=[pltpu.VMEM((bs, h), jnp.float32)],
    )
    return pl.pallas_call(
        _ffn_body,
        grid_spec=gs,
        out_shape=jax.ShapeDtypeStruct((k, s, h3), jnp.bfloat16),
        interpret=_INTERPRET,
    )(top_idx, x, W1, b1.reshape(t, 1, ff), W2, b2.reshape(t, 1, h),
      Wqkv, bqkv.reshape(1, h3))


def _attn_body(q_ref, k_ref, v_ref, o_ref, *, nkc):
    sk = k_ref.shape[0]
    kc = sk // nkc
    q = q_ref[...]
    s_chunks = []
    m = None
    for c in range(nkc):
        sc = jax.lax.dot_general(
            q, k_ref[pl.ds(c * kc, kc), :], (((1,), (1,)), ((), ())),
            preferred_element_type=jnp.float32,
        )
        s_chunks.append(sc)
        mc = jnp.max(sc, axis=-1, keepdims=True)
        m = mc if m is None else jnp.maximum(m, mc)
    acc = None
    l = None
    for c in range(nkc):
        p = jnp.exp2(s_chunks[c] - m).astype(jnp.bfloat16)
        lc = jnp.sum(p.astype(jnp.float32), axis=-1, keepdims=True)
        d = jnp.dot(p, v_ref[pl.ds(c * kc, kc), :],
                    preferred_element_type=jnp.float32)
        acc = d if acc is None else acc + d
        l = lc if l is None else l + lc
    o_ref[...] = (acc / l).astype(jnp.bfloat16)


def _attention(qkv, *, nh, dh, qbs, nkc):
    sk, h3 = qkv.shape
    h = h3 // 3
    grid = (nh, sk // qbs)
    return pl.pallas_call(
        functools.partial(_attn_body, nkc=nkc),
        grid=grid,
        in_specs=[
            pl.BlockSpec((qbs, dh), lambda hh, qb: (qb, hh)),
            pl.BlockSpec((sk, dh), lambda hh, qb: (0, nh + hh)),
            pl.BlockSpec((sk, dh), lambda hh, qb: (0, 2 * nh + hh)),
        ],
        out_specs=pl.BlockSpec((qbs, dh), lambda hh, qb: (qb, hh)),
        out_shape=jax.ShapeDtypeStruct((sk, h), jnp.bfloat16),
        interpret=_INTERPRET,
    )(qkv, qkv, qkv)


def _combine_body(w_ref, c0_ref, c1_ref, wo_ref, bo_ref, o_ref):
    mixed = w_ref[0] * c0_ref[0] + w_ref[1] * c1_ref[0]
    o_ref[...] = (
        jnp.dot(mixed, wo_ref[...], preferred_element_type=jnp.float32)
        + bo_ref[...]
    )


def _combine(ctx, w, Wo, bo, *, bs):
    k, s, h = ctx.shape
    grid = (s // bs,)
    return pl.pallas_call(
        _combine_body,
        grid=grid,
        in_specs=[
            pl.BlockSpec(memory_space=pltpu.SMEM),
            pl.BlockSpec((1, bs, h), lambda ss: (0, ss, 0)),
            pl.BlockSpec((1, bs, h), lambda ss: (1, ss, 0)),
            pl.BlockSpec((h, h), lambda ss: (0, 0)),
            pl.BlockSpec((1, h), lambda ss: (0, 0)),
        ],
        out_specs=pl.BlockSpec((bs, h), lambda ss: (ss, 0)),
        out_shape=jax.ShapeDtypeStruct((s, h), jnp.float32),
        interpret=_INTERPRET,
    )(w, ctx, ctx, Wo, bo.reshape(1, h))


def kernel(hidden_states, Wr, br, W1, b1, W2, b2, Wq, bq, Wk, bk, Wv, bv,
           Wo, bo):
    b_, s_, h_ = hidden_states.shape
    t, _, ff = W1.shape
    nh = 8
    dh = h_ // nh
    x = hidden_states.reshape(s_, h_)

    top_idx, w = _router(x, Wr, br)

    scale = float(np.log2(np.e)) / (dh ** 0.5)
    Wqkv = jnp.concatenate([Wq * scale, Wk, Wv], axis=1)
    bqkv = jnp.concatenate([bq * scale, bk, bv], axis=0)
    qkv = _ffn_qkv(x, top_idx, W1, b1, W2, b2, Wqkv, bqkv, bs=512, fblk=1024)

    k_sel = top_idx.shape[0]
    ctx = _attention(qkv.reshape(k_sel * s_, 3 * h_), nh=nh, dh=dh,
                     qbs=1024, nkc=4)

    out = _combine(ctx.reshape(k_sel, s_, h_), w, Wo, bo, bs=512)
    return out.reshape(b_, s_, h_)


# speedup vs baseline: 1.7298x; 1.0316x over previous
import functools

import jax
import jax.numpy as jnp
import numpy as np
from jax.experimental import pallas as pl
from jax.experimental.pallas import tpu as pltpu

_INTERPRET = False


def _router_body(x_ref, wr_ref, br_ref, idx_ref, w_ref):
    s = x_ref.shape[0]
    pooled = jnp.sum(x_ref[...], axis=0, keepdims=True) * (1.0 / s)
    logits = jnp.dot(pooled, wr_ref[...], preferred_element_type=jnp.float32)
    logits = logits + br_ref[...]
    m = jnp.max(logits, axis=-1, keepdims=True)
    e = jnp.exp(logits - m)
    probs = e / jnp.sum(e, axis=-1, keepdims=True)
    t = probs.shape[-1]
    iota = jax.lax.broadcasted_iota(jnp.int32, (1, t), 1)
    big = jnp.int32(2**30)
    v0 = jnp.max(probs)
    i0 = jnp.min(jnp.where(probs == v0, iota, big))
    masked = jnp.where(iota == i0, -jnp.inf, probs)
    v1 = jnp.max(masked)
    i1 = jnp.min(jnp.where(masked == v1, iota, big))
    e1 = jnp.exp(v1 - v0)
    w0 = 1.0 / (1.0 + e1)
    idx_ref[0] = i0
    idx_ref[1] = i1
    w_ref[0] = w0
    w_ref[1] = 1.0 - w0


def _router(x, Wr, br):
    t = Wr.shape[1]
    return pl.pallas_call(
        _router_body,
        out_shape=(
            jax.ShapeDtypeStruct((2,), jnp.int32),
            jax.ShapeDtypeStruct((2,), jnp.float32),
        ),
        in_specs=[
            pl.BlockSpec(memory_space=pltpu.VMEM),
            pl.BlockSpec(memory_space=pltpu.VMEM),
            pl.BlockSpec(memory_space=pltpu.VMEM),
        ],
        out_specs=(
            pl.BlockSpec(memory_space=pltpu.SMEM),
            pl.BlockSpec(memory_space=pltpu.SMEM),
        ),
        interpret=_INTERPRET,
    )(x, Wr, br.reshape(1, t))


def _ffn_body(idx_ref, x_ref, w1_ref, b1_ref, w2_ref, b2_ref, out_ref,
              acc_ref):
    f = pl.program_id(1)
    nf = pl.num_programs(1)
    h1 = jnp.dot(x_ref[...], w1_ref[0], preferred_element_type=jnp.float32)
    h1 = jax.nn.gelu(h1 + b1_ref[0])
    part = jnp.dot(h1.astype(jnp.bfloat16), w2_ref[0],
                   preferred_element_type=jnp.float32)

    @pl.when(f == 0)
    def _():
        acc_ref[...] = part + b2_ref[0]

    @pl.when(f > 0)
    def _():
        acc_ref[...] += part

    @pl.when(f == nf - 1)
    def _():
        out_ref[0] = acc_ref[...].astype(jnp.bfloat16)


def _ffn(x, top_idx, W1, b1, W2, b2, *, fblk):
    s, h = x.shape
    t, _, ff = W1.shape
    k = top_idx.shape[0]
    grid = (k, ff // fblk)
    gs = pltpu.PrefetchScalarGridSpec(
        num_scalar_prefetch=1,
        grid=grid,
        in_specs=[
            pl.BlockSpec((s, h), lambda kk, ffi, idx: (0, 0)),
            pl.BlockSpec((1, h, fblk), lambda kk, ffi, idx: (idx[kk], 0, ffi)),
            pl.BlockSpec((1, 1, fblk), lambda kk, ffi, idx: (idx[kk], 0, ffi)),
            pl.BlockSpec((1, fblk, h), lambda kk, ffi, idx: (idx[kk], ffi, 0)),
            pl.BlockSpec((1, 1, h), lambda kk, ffi, idx: (idx[kk], 0, 0)),
        ],
        out_specs=pl.BlockSpec((1, s, h), lambda kk, ffi, idx: (kk, 0, 0)),
        scratch_shapes=[pltpu.VMEM((s, h), jnp.float32)],
    )
    return pl.pallas_call(
        _ffn_body,
        grid_spec=gs,
        out_shape=jax.ShapeDtypeStruct((k, s, h), jnp.bfloat16),
        interpret=_INTERPRET,
    )(top_idx, x, W1, b1.reshape(t, 1, ff), W2, b2.reshape(t, 1, h))


def _qkv_body(x_ref, w_ref, b_ref, o_ref):
    o_ref[...] = (
        jnp.dot(x_ref[...], w_ref[...], preferred_element_type=jnp.float32)
        + b_ref[...]
    ).astype(jnp.bfloat16)


def _qkv_proj(flat, Wqkv, bqkv, *, bs):
    sk, h = flat.shape
    h3 = Wqkv.shape[1]
    return pl.pallas_call(
        _qkv_body,
        grid=(sk // bs,),
        in_specs=[
            pl.BlockSpec((bs, h), lambda ss: (ss, 0)),
            pl.BlockSpec((h, h3), lambda ss: (0, 0)),
            pl.BlockSpec((1, h3), lambda ss: (0, 0)),
        ],
        out_specs=pl.BlockSpec((bs, h3), lambda ss: (ss, 0)),
        out_shape=jax.ShapeDtypeStruct((sk, h3), jnp.bfloat16),
        interpret=_INTERPRET,
    )(flat, Wqkv, bqkv.reshape(1, h3))


def _attn_body(q_ref, k_ref, v_ref, o_ref, *, nkc):
    sk = k_ref.shape[0]
    kc = sk // nkc
    q = q_ref[...]
    s_chunks = []
    m = None
    for c in range(nkc):
        sc = jax.lax.dot_general(
            q, k_ref[pl.ds(c * kc, kc), :], (((1,), (1,)), ((), ())),
            preferred_element_type=jnp.float32,
        )
        s_chunks.append(sc)
        mc = jnp.max(sc, axis=-1, keepdims=True)
        m = mc if m is None else jnp.maximum(m, mc)
    acc = None
    l = None
    for c in range(nkc):
        p = jnp.exp2(s_chunks[c] - m).astype(jnp.bfloat16)
        lc = jnp.sum(p.astype(jnp.float32), axis=-1, keepdims=True)
        d = jnp.dot(p, v_ref[pl.ds(c * kc, kc), :],
                    preferred_element_type=jnp.float32)
        acc = d if acc is None else acc + d
        l = lc if l is None else l + lc
    o_ref[...] = (acc / l).astype(jnp.bfloat16)


def _attention(qkv, *, nh, dh, qbs, nkc):
    sk, h3 = qkv.shape
    h = h3 // 3
    grid = (nh, sk // qbs)
    return pl.pallas_call(
        functools.partial(_attn_body, nkc=nkc),
        grid=grid,
        in_specs=[
            pl.BlockSpec((qbs, dh), lambda hh, qb: (qb, hh)),
            pl.BlockSpec((sk, dh), lambda hh, qb: (0, nh + hh)),
            pl.BlockSpec((sk, dh), lambda hh, qb: (0, 2 * nh + hh)),
        ],
        out_specs=pl.BlockSpec((qbs, dh), lambda hh, qb: (qb, hh)),
        out_shape=jax.ShapeDtypeStruct((sk, h), jnp.bfloat16),
        interpret=_INTERPRET,
    )(qkv, qkv, qkv)


def _combine_body(w_ref, c0_ref, c1_ref, wo_ref, bo_ref, o_ref):
    mixed = w_ref[0] * c0_ref[0] + w_ref[1] * c1_ref[0]
    o_ref[...] = (
        jnp.dot(mixed, wo_ref[...], preferred_element_type=jnp.float32)
        + bo_ref[...]
    )


def _combine(ctx, w, Wo, bo, *, bs):
    k, s, h = ctx.shape
    grid = (s // bs,)
    return pl.pallas_call(
        _combine_body,
        grid=grid,
        in_specs=[
            pl.BlockSpec(memory_space=pltpu.SMEM),
            pl.BlockSpec((1, bs, h), lambda ss: (0, ss, 0)),
            pl.BlockSpec((1, bs, h), lambda ss: (1, ss, 0)),
            pl.BlockSpec((h, h), lambda ss: (0, 0)),
            pl.BlockSpec((1, h), lambda ss: (0, 0)),
        ],
        out_specs=pl.BlockSpec((bs, h), lambda ss: (ss, 0)),
        out_shape=jax.ShapeDtypeStruct((s, h), jnp.float32),
        interpret=_INTERPRET,
    )(w, ctx, ctx, Wo, bo.reshape(1, h))


def kernel(hidden_states, Wr, br, W1, b1, W2, b2, Wq, bq, Wk, bk, Wv, bv,
           Wo, bo):
    b_, s_, h_ = hidden_states.shape
    t, _, ff = W1.shape
    nh = 8
    dh = h_ // nh
    x = hidden_states.reshape(s_, h_)

    top_idx, w = _router(x, Wr, br)

    scale = float(np.log2(np.e)) / (dh ** 0.5)
    Wqkv = jnp.concatenate(
        [Wq * scale, Wk, Wv], axis=1).astype(jnp.bfloat16)
    bqkv = jnp.concatenate([bq * scale, bk, bv], axis=0)
    bo_out = _ffn(x, top_idx, W1, b1, W2, b2, fblk=1024)

    k_sel = top_idx.shape[0]
    qkv = _qkv_proj(bo_out.reshape(k_sel * s_, h_), Wqkv, bqkv, bs=1024)
    ctx = _attention(qkv, nh=nh, dh=dh, qbs=1024, nkc=4)

    out = _combine(ctx.reshape(k_sel, s_, h_), w, Wo, bo, bs=512)
    return out.reshape(b_, s_, h_)


# speedup vs baseline: 2.2452x; 1.2979x over previous
import functools

import jax
import jax.numpy as jnp
import numpy as np
from jax.experimental import pallas as pl
from jax.experimental.pallas import tpu as pltpu

_INTERPRET = False


def _router_body(x_ref, wr_ref, br_ref, idx_ref, w_ref):
    s = x_ref.shape[0]
    pooled = jnp.sum(x_ref[...], axis=0, keepdims=True) * (1.0 / s)
    logits = jnp.dot(pooled, wr_ref[...], preferred_element_type=jnp.float32)
    logits = logits + br_ref[...]
    m = jnp.max(logits, axis=-1, keepdims=True)
    e = jnp.exp(logits - m)
    probs = e / jnp.sum(e, axis=-1, keepdims=True)
    t = probs.shape[-1]
    iota = jax.lax.broadcasted_iota(jnp.int32, (1, t), 1)
    big = jnp.int32(2**30)
    v0 = jnp.max(probs)
    i0 = jnp.min(jnp.where(probs == v0, iota, big))
    masked = jnp.where(iota == i0, -jnp.inf, probs)
    v1 = jnp.max(masked)
    i1 = jnp.min(jnp.where(masked == v1, iota, big))
    e1 = jnp.exp(v1 - v0)
    w0 = 1.0 / (1.0 + e1)
    idx_ref[0] = i0
    idx_ref[1] = i1
    w_ref[0] = w0
    w_ref[1] = 1.0 - w0


def _router(x, Wr, br):
    t = Wr.shape[1]
    return pl.pallas_call(
        _router_body,
        out_shape=(
            jax.ShapeDtypeStruct((2,), jnp.int32),
            jax.ShapeDtypeStruct((2,), jnp.float32),
        ),
        in_specs=[
            pl.BlockSpec(memory_space=pltpu.VMEM),
            pl.BlockSpec(memory_space=pltpu.VMEM),
            pl.BlockSpec(memory_space=pltpu.VMEM),
        ],
        out_specs=(
            pl.BlockSpec(memory_space=pltpu.SMEM),
            pl.BlockSpec(memory_space=pltpu.SMEM),
        ),
        interpret=_INTERPRET,
    )(x, Wr, br.reshape(1, t))


def _ffn_body(idx_ref, x_ref, w1_ref, b1_ref, w2_ref, b2_ref, out_ref,
              acc_ref):
    f = pl.program_id(1)
    nf = pl.num_programs(1)
    h1 = jnp.dot(x_ref[...], w1_ref[0], preferred_element_type=jnp.float32)
    h1 = jax.nn.gelu(h1 + b1_ref[0])
    part = jnp.dot(h1.astype(jnp.bfloat16), w2_ref[0],
                   preferred_element_type=jnp.float32)

    @pl.when(f == 0)
    def _():
        acc_ref[...] = part + b2_ref[0]

    @pl.when(f > 0)
    def _():
        acc_ref[...] += part

    @pl.when(f == nf - 1)
    def _():
        out_ref[0] = acc_ref[...].astype(jnp.bfloat16)


def _ffn(x, top_idx, W1, b1, W2, b2, *, fblk):
    s, h = x.shape
    t, _, ff = W1.shape
    k = top_idx.shape[0]
    grid = (k, ff // fblk)
    gs = pltpu.PrefetchScalarGridSpec(
        num_scalar_prefetch=1,
        grid=grid,
        in_specs=[
            pl.BlockSpec((s, h), lambda kk, ffi, idx: (0, 0)),
            pl.BlockSpec((1, h, fblk), lambda kk, ffi, idx: (idx[kk], 0, ffi)),
            pl.BlockSpec((1, 1, fblk), lambda kk, ffi, idx: (idx[kk], 0, ffi)),
            pl.BlockSpec((1, fblk, h), lambda kk, ffi, idx: (idx[kk], ffi, 0)),
            pl.BlockSpec((1, 1, h), lambda kk, ffi, idx: (idx[kk], 0, 0)),
        ],
        out_specs=pl.BlockSpec((1, s, h), lambda kk, ffi, idx: (kk, 0, 0)),
        scratch_shapes=[pltpu.VMEM((s, h), jnp.float32)],
    )
    return pl.pallas_call(
        _ffn_body,
        grid_spec=gs,
        out_shape=jax.ShapeDtypeStruct((k, s, h), jnp.bfloat16),
        interpret=_INTERPRET,
    )(top_idx, x, W1, b1.reshape(t, 1, ff), W2, b2.reshape(t, 1, h))


def _qkv_body(x_ref, w_ref, b_ref, o_ref):
    o_ref[...] = (
        jnp.dot(x_ref[...], w_ref[...], preferred_element_type=jnp.float32)
        + b_ref[...]
    ).astype(jnp.bfloat16)


def _qkv_proj(flat, Wqkv, bqkv, *, bs):
    sk, h = flat.shape
    h3 = Wqkv.shape[1]
    return pl.pallas_call(
        _qkv_body,
        grid=(sk // bs,),
        in_specs=[
            pl.BlockSpec((bs, h), lambda ss: (ss, 0)),
            pl.BlockSpec((h, h3), lambda ss: (0, 0)),
            pl.BlockSpec((1, h3), lambda ss: (0, 0)),
        ],
        out_specs=pl.BlockSpec((bs, h3), lambda ss: (ss, 0)),
        out_shape=jax.ShapeDtypeStruct((sk, h3), jnp.bfloat16),
        interpret=_INTERPRET,
    )(flat, Wqkv, bqkv.reshape(1, h3))


def _attn_body(q_ref, k_ref, v_ref, o_ref, *, nkc):
    sk = k_ref.shape[0]
    kc = sk // nkc
    dh = q_ref.shape[1]
    q = q_ref[...]
    s_chunks = []
    m = None
    for c in range(nkc):
        sc = jax.lax.dot_general(
            q, k_ref[pl.ds(c * kc, kc), :], (((1,), (1,)), ((), ())),
            preferred_element_type=jnp.float32,
        ).astype(jnp.bfloat16)
        s_chunks.append(sc)
        mc = jnp.max(sc, axis=-1, keepdims=True)
        m = mc if m is None else jnp.maximum(m, mc)
    ones = jnp.ones((kc, dh), jnp.bfloat16)
    acc = None
    l = None
    for c in range(nkc):
        p = jnp.exp2((s_chunks[c] - m).astype(jnp.float32)).astype(jnp.bfloat16)
        d = jnp.dot(p, v_ref[pl.ds(c * kc, kc), :],
                    preferred_element_type=jnp.float32)
        lc = jnp.dot(p, ones, preferred_element_type=jnp.float32)[:, :1]
        acc = d if acc is None else acc + d
        l = lc if l is None else l + lc
    o_ref[...] = (acc / l).astype(jnp.bfloat16)


def _attention(qkv, *, nh, dh, qbs, nkc):
    sk, h3 = qkv.shape
    h = h3 // 3
    grid = (nh, sk // qbs)
    return pl.pallas_call(
        functools.partial(_attn_body, nkc=nkc),
        grid=grid,
        in_specs=[
            pl.BlockSpec((qbs, dh), lambda hh, qb: (qb, hh)),
            pl.BlockSpec((sk, dh), lambda hh, qb: (0, nh + hh)),
            pl.BlockSpec((sk, dh), lambda hh, qb: (0, 2 * nh + hh)),
        ],
        out_specs=pl.BlockSpec((qbs, dh), lambda hh, qb: (qb, hh)),
        out_shape=jax.ShapeDtypeStruct((sk, h), jnp.bfloat16),
        interpret=_INTERPRET,
    )(qkv, qkv, qkv)


def _combine_body(w_ref, c0_ref, c1_ref, wo_ref, bo_ref, o_ref):
    mixed = w_ref[0] * c0_ref[0] + w_ref[1] * c1_ref[0]
    o_ref[...] = (
        jnp.dot(mixed, wo_ref[...], preferred_element_type=jnp.float32)
        + bo_ref[...]
    )


def _combine(ctx, w, Wo, bo, *, bs):
    k, s, h = ctx.shape
    grid = (s // bs,)
    return pl.pallas_call(
        _combine_body,
        grid=grid,
        in_specs=[
            pl.BlockSpec(memory_space=pltpu.SMEM),
            pl.BlockSpec((1, bs, h), lambda ss: (0, ss, 0)),
            pl.BlockSpec((1, bs, h), lambda ss: (1, ss, 0)),
            pl.BlockSpec((h, h), lambda ss: (0, 0)),
            pl.BlockSpec((1, h), lambda ss: (0, 0)),
        ],
        out_specs=pl.BlockSpec((bs, h), lambda ss: (ss, 0)),
        out_shape=jax.ShapeDtypeStruct((s, h), jnp.float32),
        interpret=_INTERPRET,
    )(w, ctx, ctx, Wo, bo.reshape(1, h))


def kernel(hidden_states, Wr, br, W1, b1, W2, b2, Wq, bq, Wk, bk, Wv, bv,
           Wo, bo):
    b_, s_, h_ = hidden_states.shape
    t, _, ff = W1.shape
    nh = 8
    dh = h_ // nh
    x = hidden_states.reshape(s_, h_)

    top_idx, w = _router(x, Wr, br)

    scale = float(np.log2(np.e)) / (dh ** 0.5)
    Wqkv = jnp.concatenate(
        [Wq * scale, Wk, Wv], axis=1).astype(jnp.bfloat16)
    bqkv = jnp.concatenate([bq * scale, bk, bv], axis=0)
    bo_out = _ffn(x, top_idx, W1, b1, W2, b2, fblk=1024)

    k_sel = top_idx.shape[0]
    qkv = _qkv_proj(bo_out.reshape(k_sel * s_, h_), Wqkv, bqkv, bs=1024)
    ctx = _attention(qkv, nh=nh, dh=dh, qbs=1024, nkc=4)

    out = _combine(ctx.reshape(k_sel, s_, h_), w, Wo, bo, bs=512)
    return out.reshape(b_, s_, h_)


# speedup vs baseline: 2.3304x; 1.0379x over previous
import functools

import jax
import jax.numpy as jnp
import numpy as np
from jax.experimental import pallas as pl
from jax.experimental.pallas import tpu as pltpu

_INTERPRET = False


def _router_body(x_ref, wr_ref, br_ref, idx_ref, w_ref):
    s = x_ref.shape[0]
    pooled = jnp.sum(x_ref[...], axis=0, keepdims=True) * (1.0 / s)
    logits = jnp.dot(pooled, wr_ref[...], preferred_element_type=jnp.float32)
    logits = logits + br_ref[...]
    m = jnp.max(logits, axis=-1, keepdims=True)
    e = jnp.exp(logits - m)
    probs = e / jnp.sum(e, axis=-1, keepdims=True)
    t = probs.shape[-1]
    iota = jax.lax.broadcasted_iota(jnp.int32, (1, t), 1)
    big = jnp.int32(2**30)
    v0 = jnp.max(probs)
    i0 = jnp.min(jnp.where(probs == v0, iota, big))
    masked = jnp.where(iota == i0, -jnp.inf, probs)
    v1 = jnp.max(masked)
    i1 = jnp.min(jnp.where(masked == v1, iota, big))
    e1 = jnp.exp(v1 - v0)
    w0 = 1.0 / (1.0 + e1)
    idx_ref[0] = i0
    idx_ref[1] = i1
    w_ref[0] = w0
    w_ref[1] = 1.0 - w0


def _router(x, Wr, br):
    t = Wr.shape[1]
    return pl.pallas_call(
        _router_body,
        out_shape=(
            jax.ShapeDtypeStruct((2,), jnp.int32),
            jax.ShapeDtypeStruct((2,), jnp.float32),
        ),
        in_specs=[
            pl.BlockSpec(memory_space=pltpu.VMEM),
            pl.BlockSpec(memory_space=pltpu.VMEM),
            pl.BlockSpec(memory_space=pltpu.VMEM),
        ],
        out_specs=(
            pl.BlockSpec(memory_space=pltpu.SMEM),
            pl.BlockSpec(memory_space=pltpu.SMEM),
        ),
        interpret=_INTERPRET,
    )(x, Wr, br.reshape(1, t))


def _ffn_body(idx_ref, x_ref, w1_ref, b1_ref, w2_ref, b2_ref, out_ref,
              acc_ref):
    f = pl.program_id(1)
    nf = pl.num_programs(1)
    h1 = jnp.dot(x_ref[...], w1_ref[0], preferred_element_type=jnp.float32)
    h1 = jax.nn.gelu(h1 + b1_ref[0])
    part = jnp.dot(h1.astype(jnp.bfloat16), w2_ref[0],
                   preferred_element_type=jnp.float32)

    @pl.when(f == 0)
    def _():
        acc_ref[...] = part + b2_ref[0]

    @pl.when(f > 0)
    def _():
        acc_ref[...] += part

    @pl.when(f == nf - 1)
    def _():
        out_ref[0] = acc_ref[...].astype(jnp.bfloat16)


def _ffn(x, top_idx, W1, b1, W2, b2, *, fblk):
    s, h = x.shape
    t, _, ff = W1.shape
    k = top_idx.shape[0]
    grid = (k, ff // fblk)
    gs = pltpu.PrefetchScalarGridSpec(
        num_scalar_prefetch=1,
        grid=grid,
        in_specs=[
            pl.BlockSpec((s, h), lambda kk, ffi, idx: (0, 0)),
            pl.BlockSpec((1, h, fblk), lambda kk, ffi, idx: (idx[kk], 0, ffi)),
            pl.BlockSpec((1, 1, fblk), lambda kk, ffi, idx: (idx[kk], 0, ffi)),
            pl.BlockSpec((1, fblk, h), lambda kk, ffi, idx: (idx[kk], ffi, 0)),
            pl.BlockSpec((1, 1, h), lambda kk, ffi, idx: (idx[kk], 0, 0)),
        ],
        out_specs=pl.BlockSpec((1, s, h), lambda kk, ffi, idx: (kk, 0, 0)),
        scratch_shapes=[pltpu.VMEM((s, h), jnp.float32)],
    )
    return pl.pallas_call(
        _ffn_body,
        grid_spec=gs,
        out_shape=jax.ShapeDtypeStruct((k, s, h), jnp.bfloat16),
        interpret=_INTERPRET,
    )(top_idx, x, W1, b1.reshape(t, 1, ff), W2, b2.reshape(t, 1, h))


def _qkv_body(x_ref, w_ref, b_ref, o_ref):
    o_ref[...] = (
        jnp.dot(x_ref[...], w_ref[...], preferred_element_type=jnp.float32)
        + b_ref[...]
    ).astype(jnp.bfloat16)


def _qkv_proj(flat, Wqkv, bqkv, *, bs):
    sk, h = flat.shape
    h3 = Wqkv.shape[1]
    return pl.pallas_call(
        _qkv_body,
        grid=(sk // bs,),
        in_specs=[
            pl.BlockSpec((bs, h), lambda ss: (ss, 0)),
            pl.BlockSpec((h, h3), lambda ss: (0, 0)),
            pl.BlockSpec((1, h3), lambda ss: (0, 0)),
        ],
        out_specs=pl.BlockSpec((bs, h3), lambda ss: (ss, 0)),
        out_shape=jax.ShapeDtypeStruct((sk, h3), jnp.bfloat16),
        interpret=_INTERPRET,
    )(flat, Wqkv, bqkv.reshape(1, h3))


def _attn_body(q_ref, k_ref, v_ref, o_ref, *, nkc):
    sk = k_ref.shape[0]
    kc = sk // nkc
    dh = q_ref.shape[1]
    q = q_ref[...]
    s_chunks = []
    m = None
    for c in range(nkc):
        sc = jax.lax.dot_general(
            q, k_ref[pl.ds(c * kc, kc), :], (((1,), (1,)), ((), ())),
            preferred_element_type=jnp.float32,
        ).astype(jnp.bfloat16)
        s_chunks.append(sc)
        mc = jnp.max(sc, axis=-1, keepdims=True)
        m = mc if m is None else jnp.maximum(m, mc)
    ones = jnp.ones((kc, dh), jnp.bfloat16)
    acc = None
    l = None
    for c in range(nkc):
        p = jnp.exp2(s_chunks[c] - m)
        d = jnp.dot(p, v_ref[pl.ds(c * kc, kc), :],
                    preferred_element_type=jnp.float32)
        lc = jnp.dot(p, ones, preferred_element_type=jnp.float32)[:, :1]
        acc = d if acc is None else acc + d
        l = lc if l is None else l + lc
    o_ref[...] = (acc / l).astype(jnp.bfloat16)


def _attention(qkv, *, nh, dh, qbs, nkc):
    sk, h3 = qkv.shape
    h = h3 // 3
    grid = (nh, sk // qbs)
    return pl.pallas_call(
        functools.partial(_attn_body, nkc=nkc),
        grid=grid,
        in_specs=[
            pl.BlockSpec((qbs, dh), lambda hh, qb: (qb, hh)),
            pl.BlockSpec((sk, dh), lambda hh, qb: (0, nh + hh)),
            pl.BlockSpec((sk, dh), lambda hh, qb: (0, 2 * nh + hh)),
        ],
        out_specs=pl.BlockSpec((qbs, dh), lambda hh, qb: (qb, hh)),
        out_shape=jax.ShapeDtypeStruct((sk, h), jnp.bfloat16),
        interpret=_INTERPRET,
    )(qkv, qkv, qkv)


def _combine_body(w_ref, c0_ref, c1_ref, wo_ref, bo_ref, o_ref):
    mixed = w_ref[0] * c0_ref[0] + w_ref[1] * c1_ref[0]
    o_ref[...] = (
        jnp.dot(mixed, wo_ref[...], preferred_element_type=jnp.float32)
        + bo_ref[...]
    )


def _combine(ctx, w, Wo, bo, *, bs):
    k, s, h = ctx.shape
    grid = (s // bs,)
    return pl.pallas_call(
        _combine_body,
        grid=grid,
        in_specs=[
            pl.BlockSpec(memory_space=pltpu.SMEM),
            pl.BlockSpec((1, bs, h), lambda ss: (0, ss, 0)),
            pl.BlockSpec((1, bs, h), lambda ss: (1, ss, 0)),
            pl.BlockSpec((h, h), lambda ss: (0, 0)),
            pl.BlockSpec((1, h), lambda ss: (0, 0)),
        ],
        out_specs=pl.BlockSpec((bs, h), lambda ss: (ss, 0)),
        out_shape=jax.ShapeDtypeStruct((s, h), jnp.float32),
        interpret=_INTERPRET,
    )(w, ctx, ctx, Wo, bo.reshape(1, h))


def kernel(hidden_states, Wr, br, W1, b1, W2, b2, Wq, bq, Wk, bk, Wv, bv,
           Wo, bo):
    b_, s_, h_ = hidden_states.shape
    t, _, ff = W1.shape
    nh = 8
    dh = h_ // nh
    x = hidden_states.reshape(s_, h_)

    top_idx, w = _router(x, Wr, br)

    scale = float(np.log2(np.e)) / (dh ** 0.5)
    Wqkv = jnp.concatenate(
        [Wq * scale, Wk, Wv], axis=1).astype(jnp.bfloat16)
    bqkv = jnp.concatenate([bq * scale, bk, bv], axis=0)
    bo_out = _ffn(x, top_idx, W1, b1, W2, b2, fblk=1024)

    k_sel = top_idx.shape[0]
    qkv = _qkv_proj(bo_out.reshape(k_sel * s_, h_), Wqkv, bqkv, bs=1024)
    ctx = _attention(qkv, nh=nh, dh=dh, qbs=2048, nkc=8)

    out = _combine(ctx.reshape(k_sel, s_, h_), w, Wo, bo, bs=512)
    return out.reshape(b_, s_, h_)
